# baseline (speedup 1.0000x reference)
import jax
import jax.numpy as jnp
from jax import lax
from jax.experimental import pallas as pl
from jax.experimental.pallas import tpu as pltpu
from jax.experimental.pallas import tpu_sc as plsc

SEQ_LEN = 2048
OUT_DIM = 1024
BATCH = 4

NUM_CORES = 2
NUM_SUBCORES = 16
NUM_LANES = 16
NUM_WORKERS = NUM_CORES * NUM_SUBCORES
ROWS = BATCH * SEQ_LEN
SPAN = SEQ_LEN // NUM_WORKERS
CHUNK = 16
SUBWIN = SPAN // CHUNK
NCHUNK = BATCH * SUBWIN
NBUF = 3

_MESH = plsc.VectorSubcoreMesh(
    core_axis_name="c", subcore_axis_name="s",
    num_cores=NUM_CORES, num_subcores=NUM_SUBCORES)


def _sc_body(feat_hbm, pos_hbm, out_hbm, buf, posbuf, *sems):
    wid = lax.axis_index("s") * NUM_CORES + lax.axis_index("c")
    l0 = wid * SPAN
    sem_f = sems[0:NBUF]
    sem_o = sems[NBUF:2 * NBUF]
    sem_p = sems[2 * NBUF:2 * NBUF + SUBWIN]

    feat_d = [None] * NCHUNK
    out_d = [None] * NCHUNK

    def add_pos(j, h):
        @plsc.parallel_loop(0, CHUNK, 1)
        def _(r):
            row = buf.at[j, r]
            prow = posbuf.at[h * CHUNK + r]

            @plsc.parallel_loop(0, OUT_DIM, NUM_LANES, unroll=16)
            def _(ci):
                sl = pl.ds(ci, NUM_LANES)
                plsc.addupdate(row.at[sl], prow[sl])

    pos_d = [
        pltpu.async_copy(
            pos_hbm.at[pl.ds(l0 + h * CHUNK, CHUNK)],
            posbuf.at[pl.ds(h * CHUNK, CHUNK)], sem_p[h])
        for h in range(SUBWIN)
    ]

    for k in range(NCHUNK + 1):
        if k < NCHUNK:
            h, b = divmod(k, BATCH)
            j = k % NBUF
            if k >= NBUF:
                out_d[k - NBUF].wait()
            rowbase = b * SEQ_LEN + l0 + h * CHUNK
            feat_d[k] = pltpu.async_copy(
                feat_hbm.at[pl.ds(rowbase, CHUNK)], buf.at[j], sem_f[j])
        if k >= 1:
            kk = k - 1
            hh, bb = divmod(kk, BATCH)
            jj = kk % NBUF
            if bb == 0:
                pos_d[hh].wait()
            feat_d[kk].wait()
            add_pos(jj, hh)
            rowbase = bb * SEQ_LEN + l0 + hh * CHUNK
            out_d[kk] = pltpu.async_copy(
                buf.at[jj], out_hbm.at[pl.ds(rowbase, CHUNK)], sem_o[jj])

    for kk in range(NCHUNK - NBUF, NCHUNK):
        out_d[kk].wait()


def _build_sc(interpret=False):
    return pl.kernel(
        _sc_body,
        out_type=jax.ShapeDtypeStruct((ROWS, OUT_DIM), jnp.float32),
        mesh=_MESH,
        scratch_types=(
            [pltpu.VMEM((NBUF, CHUNK, OUT_DIM), jnp.float32),
             pltpu.VMEM((SPAN, OUT_DIM), jnp.float32)]
            + [pltpu.SemaphoreType.DMA] * (2 * NBUF + SUBWIN)
        ),
        interpret=interpret,
    )


_sc_pos_add = _build_sc()


def kernel(features, tokens, pos_table):
    del tokens
    B, L, D = features.shape
    out = _sc_pos_add(features.reshape(B * L, D), pos_table)
    return out.reshape(B, L, D)


# speedup vs baseline: 1.0185x; 1.0185x over previous
import jax
import jax.numpy as jnp
from jax import lax
from jax.experimental import pallas as pl
from jax.experimental.pallas import tpu as pltpu
from jax.experimental.pallas import tpu_sc as plsc

SEQ_LEN = 2048
OUT_DIM = 1024
BATCH = 4

NUM_CORES = 2
NUM_SUBCORES = 16
NUM_LANES = 16
NUM_WORKERS = NUM_CORES * NUM_SUBCORES
ROWS = BATCH * SEQ_LEN
SPAN = SEQ_LEN // NUM_WORKERS
CHUNK = 16
SUBWIN = SPAN // CHUNK
NCHUNK = BATCH * SUBWIN
NBUF = 3

_MESH = plsc.VectorSubcoreMesh(
    core_axis_name="c", subcore_axis_name="s",
    num_cores=NUM_CORES, num_subcores=NUM_SUBCORES)


def _sc_body(feat_hbm, pos_hbm, out_hbm, buf, posbuf, *sems):
    wid = lax.axis_index("s") * NUM_CORES + lax.axis_index("c")
    l0 = wid * SPAN
    sem_f = sems[0:NBUF]
    sem_o = sems[NBUF:2 * NBUF]
    sem_p = sems[2 * NBUF:2 * NBUF + SUBWIN]

    feat_d = [None] * NCHUNK
    out_d = [None] * NCHUNK

    def add_pos(j, h):
        @plsc.parallel_loop(0, CHUNK, 1)
        def _(r):
            row = buf.at[j, r]
            prow = posbuf.at[h * CHUNK + r]

            @plsc.parallel_loop(0, OUT_DIM, NUM_LANES, unroll=8)
            def _(ci):
                sl = pl.ds(ci, NUM_LANES)
                plsc.addupdate(row.at[sl], prow[sl])

    pos_d = [
        pltpu.async_copy(
            pos_hbm.at[pl.ds(l0 + h * CHUNK, CHUNK)],
            posbuf.at[pl.ds(h * CHUNK, CHUNK)], sem_p[h])
        for h in range(SUBWIN)
    ]

    for k in range(NCHUNK + 1):
        if k < NCHUNK:
            h, b = divmod(k, BATCH)
            j = k % NBUF
            if k >= NBUF:
                out_d[k - NBUF].wait()
            rowbase = b * SEQ_LEN + l0 + h * CHUNK
            feat_d[k] = pltpu.async_copy(
                feat_hbm.at[pl.ds(rowbase, CHUNK)], buf.at[j], sem_f[j])
        if k >= 1:
            kk = k - 1
            hh, bb = divmod(kk, BATCH)
            jj = kk % NBUF
            if bb == 0:
                pos_d[hh].wait()
            feat_d[kk].wait()
            add_pos(jj, hh)
            rowbase = bb * SEQ_LEN + l0 + hh * CHUNK
            out_d[kk] = pltpu.async_copy(
                buf.at[jj], out_hbm.at[pl.ds(rowbase, CHUNK)], sem_o[jj])

    for kk in range(NCHUNK - NBUF, NCHUNK):
        out_d[kk].wait()


def _build_sc(interpret=False):
    return pl.kernel(
        _sc_body,
        out_type=jax.ShapeDtypeStruct((ROWS, OUT_DIM), jnp.float32),
        mesh=_MESH,
        scratch_types=(
            [pltpu.VMEM((NBUF, CHUNK, OUT_DIM), jnp.float32),
             pltpu.VMEM((SPAN, OUT_DIM), jnp.float32)]
            + [pltpu.SemaphoreType.DMA] * (2 * NBUF + SUBWIN)
        ),
        interpret=interpret,
    )


_sc_pos_add = _build_sc()


def kernel(features, tokens, pos_table):
    del tokens
    B, L, D = features.shape
    out = _sc_pos_add(features.reshape(B * L, D), pos_table)
    return out.reshape(B, L, D)
